# baseline (speedup 1.0000x reference)
import functools

import jax
import jax.numpy as jnp
from jax import lax
from jax.experimental import pallas as pl
from jax.experimental.pallas import tpu as pltpu
from jax.experimental.pallas import tpu_sc as plsc

_BN_EPS = 1e-3
_NUM_CORES = 2
_NUM_SUBCORES = 16
_NW = _NUM_CORES * _NUM_SUBCORES
_BATCH = 16384
_DIM = 32
_BPW = _BATCH // _NW
_CHUNK = 128
_NCHUNK = _BPW // _CHUNK
_GROUP = 16
_NGROUP = _BPW // _GROUP


def _sc_body(users_hbm, items_hbm, user_table, item_table, params_hbm,
             out_hbm, idx_u, idx_i, su_v, si_v, u_rows, i_rows, pv, out_v,
             sem):
    wid = lax.axis_index("s") * _NUM_CORES + lax.axis_index("c")
    base = wid * _BPW

    pltpu.sync_copy(params_hbm, pv)
    pltpu.sync_copy(users_hbm.at[pl.ds(base, _BPW)], idx_u)
    pltpu.sync_copy(items_hbm.at[pl.ds(base, _BPW)], idx_i)

    for c in range(_BPW // 16):
        sl = pl.ds(c * 16, 16)
        ru = idx_u[sl]
        ri = idx_i[sl]
        su_v[sl] = ru & 1
        si_v[sl] = ri & 1
        idx_u[sl] = ru >> 1
        idx_i[sl] = ri >> 1

    copies = []
    for c in range(_NCHUNK):
        sl = pl.ds(c * _CHUNK, _CHUNK)
        copies.append(pltpu.async_copy(user_table.at[idx_u.at[sl]],
                                       u_rows.at[sl], sem))
        copies.append(pltpu.async_copy(item_table.at[idx_i.at[sl]],
                                       i_rows.at[sl], sem))
    for cp in copies:
        cp.wait()

    a_lo = pv[pl.ds(0, 16)]
    a_hi = pv[pl.ds(16, 16)]
    b_lo = pv[pl.ds(32, 16)]
    b_hi = pv[pl.ds(48, 16)]
    c_lo = pv[pl.ds(64, 16)]
    c_hi = pv[pl.ds(80, 16)]
    d_vec = pv[pl.ds(96, 16)]
    lanes = lax.iota(jnp.int32, 16)
    brev = (((lanes & 1) << 3) | ((lanes & 2) << 1)
            | ((lanes & 4) >> 1) | ((lanes & 8) >> 3))

    def perm(v, idx):
        return v.at[idx].get(mode="promise_in_bounds")

    def combine(a, b, x):
        m = (lanes & x) == 0
        return (jnp.where(m, a, perm(b, lanes ^ x))
                + jnp.where(m, perm(a, lanes ^ x), b))

    def group_body(g, carry):
        vs = []
        vsu = su_v[pl.ds(g * _GROUP, _GROUP)]
        vsi = si_v[pl.ds(g * _GROUP, _GROUP)]
        for j in range(_GROUP):
            k = g * _GROUP + j
            su = vsu[j]
            si = vsi[j]
            u0 = u_rows[k, su, pl.ds(0, 16)]
            u1 = u_rows[k, su, pl.ds(16, 16)]
            i0 = i_rows[k, si, pl.ds(0, 16)]
            i1 = i_rows[k, si, pl.ds(16, 16)]
            vs.append(a_lo * u0 * i0 + b_lo * u0 + c_lo * i0
                      + a_hi * u1 * i1 + b_hi * u1 + c_hi * i1)
        for x in (8, 4, 2, 1):
            vs = [combine(vs[2 * m], vs[2 * m + 1], x)
                  for m in range(len(vs) // 2)]
        out_v[pl.ds(g * _GROUP, _GROUP)] = perm(vs[0], brev) + d_vec
        return carry

    lax.fori_loop(0, _NGROUP, group_body, 0)
    pltpu.sync_copy(out_v, out_hbm.at[pl.ds(base, _BPW)])


@jax.jit
def _mf_sc(users, items, user_table, item_table, params):
    mesh = plsc.VectorSubcoreMesh(core_axis_name="c", subcore_axis_name="s",
                                  num_cores=_NUM_CORES,
                                  num_subcores=_NUM_SUBCORES)
    f = pl.kernel(
        _sc_body,
        out_type=jax.ShapeDtypeStruct((_BATCH,), jnp.float32),
        mesh=mesh,
        compiler_params=pltpu.CompilerParams(use_tc_tiling_on_sc=False),
        scratch_types=[
            pltpu.VMEM((_BPW,), jnp.int32),
            pltpu.VMEM((_BPW,), jnp.int32),
            pltpu.VMEM((_BPW,), jnp.int32),
            pltpu.VMEM((_BPW,), jnp.int32),
            pltpu.VMEM((_BPW, 2, _DIM), jnp.float32),
            pltpu.VMEM((_BPW, 2, _DIM), jnp.float32),
            pltpu.VMEM((112,), jnp.float32),
            pltpu.VMEM((_BPW,), jnp.float32),
            pltpu.SemaphoreType.DMA,
        ],
    )
    return f(users, items, user_table, item_table, params)


def kernel(users, items, user_table, item_table, gamma1, beta1, mean1, var1,
           gamma2, beta2, mean2, var2):
    a1 = gamma1 * lax.rsqrt(var1 + _BN_EPS)
    b1 = beta1 - mean1 * a1
    a2 = gamma2 * lax.rsqrt(var2 + _BN_EPS)
    b2 = beta2 - mean2 * a2
    inv = 1.0 / _DIM
    coef_a = a1 * a2 * inv
    coef_b = a1 * b2 * inv
    coef_c = a2 * b1 * inv
    coef_d = jnp.broadcast_to(jnp.sum(b1 * b2) * inv, (16,))
    params = jnp.concatenate([coef_a, coef_b, coef_c, coef_d]).astype(
        jnp.float32)
    ut3 = user_table.reshape(500000, 2, _DIM)
    it3 = item_table.reshape(500000, 2, _DIM)
    return _mf_sc(users.astype(jnp.int32), items.astype(jnp.int32),
                  ut3, it3, params)


# speedup vs baseline: 2.5911x; 2.5911x over previous
import functools

import jax
import jax.numpy as jnp
from jax import lax
from jax.experimental import pallas as pl
from jax.experimental.pallas import tpu as pltpu
from jax.experimental.pallas import tpu_sc as plsc

_BN_EPS = 1e-3
_NUM_CORES = 2
_NUM_SUBCORES = 16
_NW = _NUM_CORES * _NUM_SUBCORES
_BATCH = 16384
_DIM = 32
_TROWS = 8
_BPW = _BATCH // _NW
_CHUNK = 64
_NCHUNK = _BPW // _CHUNK
_GROUP = 16
_GPC = _CHUNK // _GROUP


def _sc_body(users_hbm, items_hbm, user_table, item_table, params_hbm,
             out_hbm, idx_u, idx_i, su_v, si_v, u_tiles, i_tiles, pv, out_v,
             sem):
    wid = lax.axis_index("s") * _NUM_CORES + lax.axis_index("c")
    base = wid * _BPW

    pltpu.sync_copy(params_hbm, pv)
    pltpu.sync_copy(users_hbm.at[pl.ds(base, _BPW)], idx_u)
    pltpu.sync_copy(items_hbm.at[pl.ds(base, _BPW)], idx_i)

    for c in range(_BPW // 16):
        sl = pl.ds(c * 16, 16)
        ru = idx_u[sl]
        ri = idx_i[sl]
        su_v[sl] = ru & 7
        si_v[sl] = ri & 7
        idx_u[sl] = ru >> 3
        idx_i[sl] = ri >> 3

    a_lo = pv[pl.ds(0, 16)]
    a_hi = pv[pl.ds(16, 16)]
    b_lo = pv[pl.ds(32, 16)]
    b_hi = pv[pl.ds(48, 16)]
    c_lo = pv[pl.ds(64, 16)]
    c_hi = pv[pl.ds(80, 16)]
    d_vec = pv[pl.ds(96, 16)]
    lanes = lax.iota(jnp.int32, 16)
    brev = (((lanes & 1) << 3) | ((lanes & 2) << 1)
            | ((lanes & 4) >> 1) | ((lanes & 8) >> 3))

    def perm(v, idx):
        return v.at[idx].get(mode="promise_in_bounds")

    def combine(a, b, x):
        m = (lanes & x) == 0
        return (jnp.where(m, a, perm(b, lanes ^ x))
                + jnp.where(m, perm(a, lanes ^ x), b))

    for c in range(_NCHUNK):
        sl = pl.ds(c * _CHUNK, _CHUNK)
        cu = pltpu.async_copy(user_table.at[idx_u.at[sl]], u_tiles, sem)
        ci = pltpu.async_copy(item_table.at[idx_i.at[sl]], i_tiles, sem)
        cu.wait()
        ci.wait()

        def group_body(gg, carry, c=c):
            g = c * _GPC + gg
            vs = []
            vsu = su_v[pl.ds(g * _GROUP, _GROUP)]
            vsi = si_v[pl.ds(g * _GROUP, _GROUP)]
            for j in range(_GROUP):
                kk = gg * _GROUP + j
                su = vsu[j]
                si = vsi[j]
                u0 = u_tiles[kk, su, pl.ds(0, 16)]
                u1 = u_tiles[kk, su, pl.ds(16, 16)]
                i0 = i_tiles[kk, si, pl.ds(0, 16)]
                i1 = i_tiles[kk, si, pl.ds(16, 16)]
                vs.append(a_lo * u0 * i0 + b_lo * u0 + c_lo * i0
                          + a_hi * u1 * i1 + b_hi * u1 + c_hi * i1)
            for x in (8, 4, 2, 1):
                vs = [combine(vs[2 * m], vs[2 * m + 1], x)
                      for m in range(len(vs) // 2)]
            out_v[pl.ds(g * _GROUP, _GROUP)] = perm(vs[0], brev) + d_vec
            return carry

        lax.fori_loop(0, _GPC, group_body, 0)

    pltpu.sync_copy(out_v, out_hbm.at[pl.ds(base, _BPW)])


@jax.jit
def _mf_sc(users, items, user_table, item_table, params):
    mesh = plsc.VectorSubcoreMesh(core_axis_name="c", subcore_axis_name="s",
                                  num_cores=_NUM_CORES,
                                  num_subcores=_NUM_SUBCORES)
    f = pl.kernel(
        _sc_body,
        out_type=jax.ShapeDtypeStruct((_BATCH,), jnp.float32),
        mesh=mesh,
        compiler_params=pltpu.CompilerParams(use_tc_tiling_on_sc=False),
        scratch_types=[
            pltpu.VMEM((_BPW,), jnp.int32),
            pltpu.VMEM((_BPW,), jnp.int32),
            pltpu.VMEM((_BPW,), jnp.int32),
            pltpu.VMEM((_BPW,), jnp.int32),
            pltpu.VMEM((_CHUNK, _TROWS, _DIM), jnp.float32),
            pltpu.VMEM((_CHUNK, _TROWS, _DIM), jnp.float32),
            pltpu.VMEM((112,), jnp.float32),
            pltpu.VMEM((_BPW,), jnp.float32),
            pltpu.SemaphoreType.DMA,
        ],
    )
    return f(users, items, user_table, item_table, params)


def kernel(users, items, user_table, item_table, gamma1, beta1, mean1, var1,
           gamma2, beta2, mean2, var2):
    a1 = gamma1 * lax.rsqrt(var1 + _BN_EPS)
    b1 = beta1 - mean1 * a1
    a2 = gamma2 * lax.rsqrt(var2 + _BN_EPS)
    b2 = beta2 - mean2 * a2
    inv = 1.0 / _DIM
    coef_a = a1 * a2 * inv
    coef_b = a1 * b2 * inv
    coef_c = a2 * b1 * inv
    coef_d = jnp.broadcast_to(jnp.sum(b1 * b2) * inv, (16,))
    params = jnp.concatenate([coef_a, coef_b, coef_c, coef_d]).astype(
        jnp.float32)
    ut3 = user_table.reshape(1000000 // _TROWS, _TROWS, _DIM)
    it3 = item_table.reshape(1000000 // _TROWS, _TROWS, _DIM)
    return _mf_sc(users.astype(jnp.int32), items.astype(jnp.int32),
                  ut3, it3, params)


# speedup vs baseline: 6.0401x; 2.3311x over previous
import functools

import jax
import jax.numpy as jnp
from jax import lax
from jax.experimental import pallas as pl
from jax.experimental.pallas import tpu as pltpu
from jax.experimental.pallas import tpu_sc as plsc

_BN_EPS = 1e-3
_NUM_CORES = 2
_NUM_SUBCORES = 16
_NW = _NUM_CORES * _NUM_SUBCORES
_BATCH = 16384
_DIM = 32
_TROWS = 8
_BPW = _BATCH // _NW
_GROUP = 16
_NGROUP = _BPW // _GROUP


def _sc_body(users_hbm, items_hbm, user_table, item_table, params_hbm,
             out_hbm, sub_u, sub_i,
             u_tiles, i_tiles, pv, out_v, sem_a, sem_b):
    wid = lax.axis_index("s") * _NUM_CORES + lax.axis_index("c")
    base = wid * _BPW

    pltpu.sync_copy(params_hbm, pv)
    pltpu.sync_copy(users_hbm.at[pl.ds(base, _BPW)], sub_u)
    pltpu.sync_copy(items_hbm.at[pl.ds(base, _BPW)], sub_i)

    def fire(g, par, sem):
        vu = sub_u[pl.ds(g * _GROUP, _GROUP)] >> 3
        vi = sub_i[pl.ds(g * _GROUP, _GROUP)] >> 3
        for j in range(_GROUP):
            pltpu.async_copy(user_table.at[vu[j]], u_tiles.at[par, j], sem)
            pltpu.async_copy(item_table.at[vi[j]], i_tiles.at[par, j], sem)

    def drain(par, sem):
        pltpu.make_async_copy(user_table.at[pl.ds(0, _GROUP)],
                              u_tiles.at[par], sem).wait()
        pltpu.make_async_copy(item_table.at[pl.ds(0, _GROUP)],
                              i_tiles.at[par], sem).wait()


    a_lo = pv[pl.ds(0, 16)]
    a_hi = pv[pl.ds(16, 16)]
    b_lo = pv[pl.ds(32, 16)]
    b_hi = pv[pl.ds(48, 16)]
    c_lo = pv[pl.ds(64, 16)]
    c_hi = pv[pl.ds(80, 16)]
    d_vec = pv[pl.ds(96, 16)]
    lanes = lax.iota(jnp.int32, 16)
    brev = (((lanes & 1) << 3) | ((lanes & 2) << 1)
            | ((lanes & 4) >> 1) | ((lanes & 8) >> 3))

    def perm(v, idx):
        return v.at[idx].get(mode="promise_in_bounds")

    def combine(a, b, x):
        m = (lanes & x) == 0
        return (jnp.where(m, a, perm(b, lanes ^ x))
                + jnp.where(m, perm(a, lanes ^ x), b))

    fire(0, 0, sem_a)

    def group_body(g, carry):
        par = lax.rem(g, 2)

        @pl.when(g + 1 < _NGROUP)
        def _prefetch():
            @pl.when(par == 0)
            def _():
                fire(g + 1, 1, sem_b)

            @pl.when(par == 1)
            def _():
                fire(g + 1, 0, sem_a)

        @pl.when(par == 0)
        def _():
            drain(0, sem_a)

        @pl.when(par == 1)
        def _():
            drain(1, sem_b)

        vs = []
        vsu = sub_u[pl.ds(g * _GROUP, _GROUP)] & 7
        vsi = sub_i[pl.ds(g * _GROUP, _GROUP)] & 7
        for j in range(_GROUP):
            su = vsu[j]
            si = vsi[j]
            u0 = u_tiles[par, j, su, pl.ds(0, 16)]
            u1 = u_tiles[par, j, su, pl.ds(16, 16)]
            i0 = i_tiles[par, j, si, pl.ds(0, 16)]
            i1 = i_tiles[par, j, si, pl.ds(16, 16)]
            vs.append(a_lo * u0 * i0 + b_lo * u0 + c_lo * i0
                      + a_hi * u1 * i1 + b_hi * u1 + c_hi * i1)
        for x in (8, 4, 2, 1):
            vs = [combine(vs[2 * m], vs[2 * m + 1], x)
                  for m in range(len(vs) // 2)]
        out_v[pl.ds(g * _GROUP, _GROUP)] = perm(vs[0], brev) + d_vec
        return carry

    lax.fori_loop(0, _NGROUP, group_body, 0)
    pltpu.sync_copy(out_v, out_hbm.at[pl.ds(base, _BPW)])


@jax.jit
def _mf_sc(users, items, user_table, item_table, params):
    mesh = plsc.VectorSubcoreMesh(core_axis_name="c", subcore_axis_name="s",
                                  num_cores=_NUM_CORES,
                                  num_subcores=_NUM_SUBCORES)
    f = pl.kernel(
        _sc_body,
        out_type=jax.ShapeDtypeStruct((_BATCH,), jnp.float32),
        mesh=mesh,
        compiler_params=pltpu.CompilerParams(use_tc_tiling_on_sc=True),
        scratch_types=[
            pltpu.VMEM((_BPW,), jnp.int32),
            pltpu.VMEM((_BPW,), jnp.int32),
            pltpu.VMEM((2, _GROUP, _TROWS, _DIM), jnp.float32),
            pltpu.VMEM((2, _GROUP, _TROWS, _DIM), jnp.float32),
            pltpu.VMEM((112,), jnp.float32),
            pltpu.VMEM((_BPW,), jnp.float32),
            pltpu.SemaphoreType.DMA,
            pltpu.SemaphoreType.DMA,
        ],
    )
    return f(users, items, user_table, item_table, params)


def kernel(users, items, user_table, item_table, gamma1, beta1, mean1, var1,
           gamma2, beta2, mean2, var2):
    a1 = gamma1 * lax.rsqrt(var1 + _BN_EPS)
    b1 = beta1 - mean1 * a1
    a2 = gamma2 * lax.rsqrt(var2 + _BN_EPS)
    b2 = beta2 - mean2 * a2
    inv = 1.0 / _DIM
    coef_a = a1 * a2 * inv
    coef_b = a1 * b2 * inv
    coef_c = a2 * b1 * inv
    coef_d = jnp.broadcast_to(jnp.sum(b1 * b2) * inv, (16,))
    params = jnp.concatenate([coef_a, coef_b, coef_c, coef_d]).astype(
        jnp.float32)
    ut3 = user_table.reshape(1000000 // _TROWS, _TROWS, _DIM)
    it3 = item_table.reshape(1000000 // _TROWS, _TROWS, _DIM)
    return _mf_sc(users.astype(jnp.int32), items.astype(jnp.int32),
                  ut3, it3, params)
